# baseline (speedup 1.0000x reference)
import jax
import jax.numpy as jnp
from jax.experimental import pallas as pl
from jax.experimental.pallas import tpu as pltpu


def _gcn_fused_kernel(x_ref, w_ref, a_lo_ref, a_hi_ref, o_lo_ref, o_hi_ref, xw_ref):
    @pl.when(pl.program_id(0) == 0)
    def _():
        xw_ref[...] = jnp.dot(
            x_ref[...], w_ref[...], preferred_element_type=jnp.float32
        )

    o_lo_ref[...] = jnp.dot(
        a_lo_ref[...], xw_ref[...], preferred_element_type=jnp.float32
    )
    o_hi_ref[...] = jnp.dot(
        a_hi_ref[...], xw_ref[...], preferred_element_type=jnp.float32
    )


def kernel(X, A_hat, W):
    n, d_in = X.shape
    d_out = W.shape[1]
    bm = 200
    steps = (n // 2) // bm
    lo, hi = pl.pallas_call(
        _gcn_fused_kernel,
        grid=(steps,),
        in_specs=[
            pl.BlockSpec((n, d_in), lambda m: (0, 0)),
            pl.BlockSpec((d_in, d_out), lambda m: (0, 0)),
            pl.BlockSpec((bm, n), lambda m: (m, 0)),
            pl.BlockSpec((bm, n), lambda m: (m + steps, 0)),
        ],
        out_specs=[
            pl.BlockSpec((bm, d_out), lambda m: (m, 0)),
            pl.BlockSpec((bm, d_out), lambda m: (m, 0)),
        ],
        out_shape=[
            jax.ShapeDtypeStruct((n // 2, d_out), jnp.float32),
            jax.ShapeDtypeStruct((n // 2, d_out), jnp.float32),
        ],
        scratch_shapes=[pltpu.VMEM((n, d_out), jnp.float32)],
        compiler_params=pltpu.CompilerParams(
            dimension_semantics=("arbitrary",),
        ),
    )(X, W, A_hat, A_hat)
    return jnp.concatenate([lo, hi], axis=0)


# speedup vs baseline: 1.0099x; 1.0099x over previous
import jax
import jax.numpy as jnp
from jax.experimental import pallas as pl
from jax.experimental.pallas import tpu as pltpu


def _xw_kernel(x_ref, w_ref, o_ref):
    o_ref[...] = jnp.dot(
        x_ref[...], w_ref[...], preferred_element_type=jnp.float32
    )


def _spmm_kernel(xw_ref, a_ref, o_ref):
    o_ref[...] = jnp.dot(
        a_ref[...], xw_ref[...], preferred_element_type=jnp.float32
    )


def kernel(X, A_hat, W):
    n, d_in = X.shape
    d_out = W.shape[1]
    xw = pl.pallas_call(
        _xw_kernel,
        out_shape=jax.ShapeDtypeStruct((n, d_out), jnp.float32),
    )(X, W)
    bm = 400
    return pl.pallas_call(
        _spmm_kernel,
        grid=(n // bm,),
        in_specs=[
            pl.BlockSpec((n, d_out), lambda m: (0, 0)),
            pl.BlockSpec((bm, n), lambda m: (m, 0)),
        ],
        out_specs=pl.BlockSpec((bm, d_out), lambda m: (m, 0)),
        out_shape=jax.ShapeDtypeStruct((n, d_out), jnp.float32),
        compiler_params=pltpu.CompilerParams(
            dimension_semantics=("parallel",),
        ),
    )(xw, A_hat)


# speedup vs baseline: 1.0465x; 1.0363x over previous
import jax
import jax.numpy as jnp
from jax.experimental import pallas as pl
from jax.experimental.pallas import tpu as pltpu


def _gcn_fused_kernel(x_ref, w_ref, a_ref, o_ref, xw_ref):
    @pl.when(pl.program_id(0) == 0)
    def _():
        xw_ref[...] = jnp.dot(
            x_ref[...], w_ref[...], preferred_element_type=jnp.float32
        )

    o_ref[...] = jnp.dot(
        a_ref[...], xw_ref[...], preferred_element_type=jnp.float32
    )


def kernel(X, A_hat, W):
    n, d_in = X.shape
    d_out = W.shape[1]
    bm = 400
    return pl.pallas_call(
        _gcn_fused_kernel,
        grid=(n // bm,),
        in_specs=[
            pl.BlockSpec((n, d_in), lambda m: (0, 0),
                         pipeline_mode=pl.Buffered(buffer_count=1)),
            pl.BlockSpec((d_in, d_out), lambda m: (0, 0),
                         pipeline_mode=pl.Buffered(buffer_count=1)),
            pl.BlockSpec((bm, n), lambda m: (m, 0)),
        ],
        out_specs=pl.BlockSpec((bm, d_out), lambda m: (m, 0)),
        out_shape=jax.ShapeDtypeStruct((n, d_out), jnp.float32),
        scratch_shapes=[pltpu.VMEM((n, d_out), jnp.float32)],
        compiler_params=pltpu.CompilerParams(
            dimension_semantics=("arbitrary",),
            vmem_limit_bytes=128 * 1024 * 1024,
        ),
    )(X, W, A_hat)


# speedup vs baseline: 1.0543x; 1.0074x over previous
import jax
import jax.numpy as jnp
from jax.experimental import pallas as pl
from jax.experimental.pallas import tpu as pltpu


def _gcn_fused_kernel(x_ref, w_ref, a_ref, o_ref, xw_ref):
    @pl.when(pl.program_id(0) == 0)
    def _():
        xw_ref[...] = jnp.dot(
            x_ref[...], w_ref[...], preferred_element_type=jnp.float32
        )

    o_ref[...] = jnp.dot(
        a_ref[...], xw_ref[...], preferred_element_type=jnp.float32
    )


def kernel(X, A_hat, W):
    n, d_in = X.shape
    d_out = W.shape[1]
    bm = 400
    return pl.pallas_call(
        _gcn_fused_kernel,
        grid=(n // bm,),
        in_specs=[
            pl.BlockSpec((n, d_in), lambda m: (0, 0)),
            pl.BlockSpec((d_in, d_out), lambda m: (0, 0)),
            pl.BlockSpec((bm, n), lambda m: (m, 0)),
        ],
        out_specs=pl.BlockSpec((bm, d_out), lambda m: (m, 0)),
        out_shape=jax.ShapeDtypeStruct((n, d_out), jnp.float32),
        scratch_shapes=[pltpu.VMEM((n, d_out), jnp.float32)],
        compiler_params=pltpu.CompilerParams(
            dimension_semantics=("arbitrary",),
        ),
    )(X, W, A_hat)
